# baseline (speedup 1.0000x reference)
import functools

import jax
import jax.numpy as jnp
from jax import lax
from jax.experimental import pallas as pl
from jax.experimental.pallas import tpu as pltpu
from jax.experimental.pallas import tpu_sc as plsc

_ABS_MASK = 0x7FFFFFFF
_NWORKERS = 32
_WIN = 8192


def _sparsity() -> float:
    s_i, s_f = 0.5, 0.9
    return s_f + (s_i - s_f) * (1.0 - 0.0) ** 3


def _make_sc_hist(numel, nbuckets, shift, bmask, filt_shift):
    shard = numel // _NWORKERS
    nwin = shard // _WIN
    hwords = 16 * nbuckets
    mesh = plsc.VectorSubcoreMesh(
        core_axis_name="c", subcore_axis_name="s", num_cores=2, num_subcores=16
    )

    def body(x_hbm, *rest):
        if filt_shift is not None:
            sel_hbm, hist_hbm, wbuf, hist_v, scal_v, sem0, sem1 = rest
        else:
            hist_hbm, wbuf, hist_v, scal_v, sem0, sem1 = rest
        sems = (sem0, sem1)
        cid = lax.axis_index("c")
        sid = lax.axis_index("s")
        wid = sid * 2 + cid
        base = wid * shard

        zero16 = jnp.zeros((16,), jnp.int32)

        def zbody(i, _):
            hist_v[pl.ds(i * 16, 16)] = zero16
            return 0

        lax.fori_loop(0, hwords // 16, zbody, 0, unroll=8)

        if filt_shift is not None:
            pltpu.sync_copy(sel_hbm, scal_v)
            selv = scal_v[...]

        lanebase = lax.iota(jnp.int32, 16) * nbuckets
        ones = jnp.ones((16,), jnp.int32)

        def start_copy(g, b):
            pltpu.make_async_copy(
                x_hbm.at[pl.ds(base + g * _WIN, _WIN)], wbuf.at[b], sems[b]
            ).start()

        def wait_copy(g, b):
            pltpu.make_async_copy(
                x_hbm.at[pl.ds(base + g * _WIN, _WIN)], wbuf.at[b], sems[b]
            ).wait()

        def window(g, b):
            @pl.when(g + 1 < nwin)
            def _():
                start_copy(g + 1, 1 - b)

            wait_copy(g, b)

            def compute(j, _):
                for i in range(64):
                    off = j * 1024 + i * 16
                    bits = wbuf[b, pl.ds(off, 16)] & jnp.int32(_ABS_MASK)
                    bucket = bits
                    if shift:
                        bucket = lax.shift_right_logical(bucket, shift)
                    if bmask is not None:
                        bucket = bucket & jnp.int32(bmask)
                    idx = lanebase + bucket
                    if filt_shift is not None:
                        m = lax.shift_right_logical(bits, filt_shift) == selv
                    else:
                        m = bits == bits
                    plsc.addupdate_scatter(hist_v, [idx], ones, mask=m)
                return 0

            lax.fori_loop(0, _WIN // 1024, compute, 0)

        def pair(gp, _):
            for b in range(2):
                window(gp * 2 + b, b)
            return 0

        start_copy(0, 0)
        lax.fori_loop(0, nwin // 2, pair, 0)
        pltpu.sync_copy(hist_v, hist_hbm.at[wid])

    return functools.partial(
        pl.kernel,
        body,
        out_type=jax.ShapeDtypeStruct((_NWORKERS, hwords), jnp.int32),
        mesh=mesh,
        compiler_params=pltpu.CompilerParams(needs_layout_passes=False),
        scratch_types=[
            pltpu.VMEM((2, _WIN), jnp.int32),
            pltpu.VMEM((hwords,), jnp.int32),
            pltpu.VMEM((16,), jnp.int32),
            pltpu.SemaphoreType.DMA,
            pltpu.SemaphoreType.DMA,
        ],
    )()


def _make_merge(nbuckets, shift_this, k_static):
    nbits = nbuckets.bit_length() - 1
    has_prev = k_static is None

    def body(*refs):
        if has_prev:
            psel_ref, pkrem_ref, hist_ref, sel_out, krem_out = refs
            prev_sel = psel_ref[0]
            k_t = pkrem_ref[0]
        else:
            hist_ref, sel_out, krem_out = refs
            prev_sel = jnp.int32(0)
            k_t = jnp.int32(k_static)
        h = jnp.sum(hist_ref[...], axis=0, keepdims=True)
        iot = lax.broadcasted_iota(jnp.int32, (1, nbuckets), 1)
        lo = jnp.int32(0)
        hi = jnp.int32(nbuckets)
        for _ in range(nbits):
            mid = (lo + hi) // 2
            c = jnp.sum(jnp.where(iot >= mid, h, 0))
            ge = c >= k_t
            lo = jnp.where(ge, mid, lo)
            hi = jnp.where(ge, hi, mid)
        c_above = jnp.sum(jnp.where(iot >= lo + 1, h, 0))
        new_sel = (prev_sel * jnp.int32(nbuckets)) + lo
        for i in range(16):
            sel_out[i] = new_sel
        krem_out[0] = k_t - c_above

    in_specs = [pl.BlockSpec(memory_space=pltpu.VMEM)]
    if has_prev:
        in_specs = [
            pl.BlockSpec(memory_space=pltpu.SMEM),
            pl.BlockSpec(memory_space=pltpu.SMEM),
        ] + in_specs
    return pl.pallas_call(
        body,
        in_specs=in_specs,
        out_specs=[
            pl.BlockSpec(memory_space=pltpu.SMEM),
            pl.BlockSpec(memory_space=pltpu.SMEM),
        ],
        out_shape=[
            jax.ShapeDtypeStruct((16,), jnp.int32),
            jax.ShapeDtypeStruct((1,), jnp.int32),
        ],
    )


def _mask_body(t_ref, x_ref, o_ref):
    t = t_ref[0]
    xv = x_ref[...]
    bits = lax.bitcast_convert_type(xv, jnp.int32) & jnp.int32(_ABS_MASK)
    o_ref[...] = jnp.where(bits >= t, xv, 0.0)


def kernel(x, bias):
    rows, cols = x.shape
    numel = rows * cols
    k = max(1, int(round(numel * (1.0 - _sparsity()))))
    xf = lax.bitcast_convert_type(x, jnp.int32).reshape(-1)

    hist1 = _make_sc_hist(numel, 4096, 19, None, None)(xf)
    sel1, krem1 = _make_merge(4096, 12, k)(hist1.reshape(_NWORKERS * 16, 4096))

    hist2 = _make_sc_hist(numel, 4096, 7, 0xFFF, 19)(xf, sel1)
    sel2, krem2 = _make_merge(4096, 12, None)(
        sel1, krem1, hist2.reshape(_NWORKERS * 16, 4096)
    )

    hist3 = _make_sc_hist(numel, 128, 0, 0x7F, 7)(xf, sel2)
    sel3, _ = _make_merge(128, 7, None)(
        sel2, krem2, hist3.reshape(_NWORKERS * 16, 128)
    )

    nchunks = 8
    blk = rows // nchunks
    masked = pl.pallas_call(
        _mask_body,
        grid=(nchunks,),
        in_specs=[
            pl.BlockSpec(memory_space=pltpu.SMEM),
            pl.BlockSpec((blk, cols), lambda c: (c, 0)),
        ],
        out_specs=pl.BlockSpec((blk, cols), lambda c: (c, 0)),
        out_shape=jax.ShapeDtypeStruct((rows, cols), jnp.float32),
    )(sel3, x)

    return (masked, bias)


# speedup vs baseline: 1.0245x; 1.0245x over previous
import functools

import jax
import jax.numpy as jnp
from jax import lax
from jax.experimental import pallas as pl
from jax.experimental.pallas import tpu as pltpu
from jax.experimental.pallas import tpu_sc as plsc

_ABS_MASK = 0x7FFFFFFF
_NWORKERS = 32
_WIN = 8192


def _sparsity() -> float:
    s_i, s_f = 0.5, 0.9
    return s_f + (s_i - s_f) * (1.0 - 0.0) ** 3


def _make_sc_hist(numel, nbuckets, shift, filt_shift):
    shard = numel // _NWORKERS
    nwin = shard // _WIN
    hwords = 16 * nbuckets
    idx_mask = (nbuckets - 1) << shift
    mesh = plsc.VectorSubcoreMesh(
        core_axis_name="c", subcore_axis_name="s", num_cores=2, num_subcores=16
    )

    def body(x_hbm, *rest):
        if filt_shift is not None:
            sel_hbm, hist_hbm, wbuf, hist_v, scal_v, sem0, sem1 = rest
        else:
            hist_hbm, wbuf, hist_v, scal_v, sem0, sem1 = rest
        sems = (sem0, sem1)
        cid = lax.axis_index("c")
        sid = lax.axis_index("s")
        wid = sid * 2 + cid
        base = wid * shard

        zero16 = jnp.zeros((16,), jnp.int32)

        def zbody(i, _):
            hist_v[pl.ds(i * 16, 16)] = zero16
            return 0

        lax.fori_loop(0, hwords // 16, zbody, 0, unroll=8)

        if filt_shift is not None:
            pltpu.sync_copy(sel_hbm, scal_v)
            selv = scal_v[...]

        lane = lax.iota(jnp.int32, 16)
        ones = jnp.ones((16,), jnp.int32)

        def start_copy(g, b):
            pltpu.make_async_copy(
                x_hbm.at[pl.ds(base + g * _WIN, _WIN)], wbuf.at[b], sems[b]
            ).start()

        def wait_copy(g, b):
            pltpu.make_async_copy(
                x_hbm.at[pl.ds(base + g * _WIN, _WIN)], wbuf.at[b], sems[b]
            ).wait()

        def window(g, b):
            @pl.when(g + 1 < nwin)
            def _():
                start_copy(g + 1, 1 - b)

            wait_copy(g, b)

            def compute(j, _):
                for i in range(64):
                    off = j * 1024 + i * 16
                    raw = wbuf[b, pl.ds(off, 16)]
                    masked = raw & jnp.int32(idx_mask)
                    if shift >= 4:
                        bkt16 = lax.shift_right_logical(masked, shift - 4)
                    else:
                        bkt16 = lax.shift_left(masked, 4 - shift)
                    idx = bkt16 | lane
                    if filt_shift is not None:
                        bits = raw & jnp.int32(_ABS_MASK)
                        m = lax.shift_right_logical(bits, filt_shift) == selv
                        plsc.addupdate_scatter(hist_v, [idx], ones, mask=m)
                    else:
                        plsc.addupdate_scatter(hist_v, [idx], ones)
                return 0

            lax.fori_loop(0, _WIN // 1024, compute, 0)

        def pair(gp, _):
            for b in range(2):
                window(gp * 2 + b, b)
            return 0

        start_copy(0, 0)
        lax.fori_loop(0, nwin // 2, pair, 0)
        pltpu.sync_copy(hist_v, hist_hbm.at[wid])

    return functools.partial(
        pl.kernel,
        body,
        out_type=jax.ShapeDtypeStruct((_NWORKERS, hwords), jnp.int32),
        mesh=mesh,
        compiler_params=pltpu.CompilerParams(needs_layout_passes=False),
        scratch_types=[
            pltpu.VMEM((2, _WIN), jnp.int32),
            pltpu.VMEM((hwords,), jnp.int32),
            pltpu.VMEM((16,), jnp.int32),
            pltpu.SemaphoreType.DMA,
            pltpu.SemaphoreType.DMA,
        ],
    )()


def _make_merge(nbuckets, shift_this, k_static):
    nbits = nbuckets.bit_length() - 1
    has_prev = k_static is None

    def body(*refs):
        if has_prev:
            psel_ref, pkrem_ref, hist_ref, sel_out, krem_out = refs
            prev_sel = psel_ref[0]
            k_t = pkrem_ref[0]
        else:
            hist_ref, sel_out, krem_out = refs
            prev_sel = jnp.int32(0)
            k_t = jnp.int32(k_static)
        h = jnp.sum(hist_ref[...], axis=0, keepdims=True)
        iot = lax.broadcasted_iota(jnp.int32, (1, nbuckets * 16), 1)
        lo = jnp.int32(0)
        hi = jnp.int32(nbuckets)
        for _ in range(nbits):
            mid = (lo + hi) // 2
            c = jnp.sum(jnp.where(iot >= mid * 16, h, 0))
            ge = c >= k_t
            lo = jnp.where(ge, mid, lo)
            hi = jnp.where(ge, hi, mid)
        c_above = jnp.sum(jnp.where(iot >= (lo + 1) * 16, h, 0))
        new_sel = (prev_sel * jnp.int32(nbuckets)) + lo
        for i in range(16):
            sel_out[i] = new_sel
        krem_out[0] = k_t - c_above

    in_specs = [pl.BlockSpec(memory_space=pltpu.VMEM)]
    if has_prev:
        in_specs = [
            pl.BlockSpec(memory_space=pltpu.SMEM),
            pl.BlockSpec(memory_space=pltpu.SMEM),
        ] + in_specs
    return pl.pallas_call(
        body,
        in_specs=in_specs,
        out_specs=[
            pl.BlockSpec(memory_space=pltpu.SMEM),
            pl.BlockSpec(memory_space=pltpu.SMEM),
        ],
        out_shape=[
            jax.ShapeDtypeStruct((16,), jnp.int32),
            jax.ShapeDtypeStruct((1,), jnp.int32),
        ],
    )


def _mask_body(t_ref, x_ref, o_ref):
    t = t_ref[0]
    xv = x_ref[...]
    bits = lax.bitcast_convert_type(xv, jnp.int32) & jnp.int32(_ABS_MASK)
    o_ref[...] = jnp.where(bits >= t, xv, 0.0)


def kernel(x, bias):
    rows, cols = x.shape
    numel = rows * cols
    k = max(1, int(round(numel * (1.0 - _sparsity()))))
    xf = lax.bitcast_convert_type(x, jnp.int32).reshape(-1)

    hist1 = _make_sc_hist(numel, 4096, 19, None)(xf)
    sel1, krem1 = _make_merge(4096, 12, k)(hist1)

    hist2 = _make_sc_hist(numel, 4096, 7, 19)(xf, sel1)
    sel2, krem2 = _make_merge(4096, 12, None)(sel1, krem1, hist2)

    hist3 = _make_sc_hist(numel, 128, 0, 7)(xf, sel2)
    sel3, _ = _make_merge(128, 7, None)(sel2, krem2, hist3)

    nchunks = 8
    blk = rows // nchunks
    masked = pl.pallas_call(
        _mask_body,
        grid=(nchunks,),
        in_specs=[
            pl.BlockSpec(memory_space=pltpu.SMEM),
            pl.BlockSpec((blk, cols), lambda c: (c, 0)),
        ],
        out_specs=pl.BlockSpec((blk, cols), lambda c: (c, 0)),
        out_shape=jax.ShapeDtypeStruct((rows, cols), jnp.float32),
    )(sel3, x)

    return (masked, bias)


# speedup vs baseline: 2.8189x; 2.7515x over previous
import functools

import jax
import jax.numpy as jnp
from jax import lax
from jax.experimental import pallas as pl
from jax.experimental.pallas import tpu as pltpu
from jax.experimental.pallas import tpu_sc as plsc

_ABS_MASK = 0x7FFFFFFF
_NWORKERS = 32
_WIN = 8192


def _sparsity() -> float:
    s_i, s_f = 0.5, 0.9
    return s_f + (s_i - s_f) * (1.0 - 0.0) ** 3


def _make_sc_hist(numel, nbuckets, shift, filt_shift):
    shard = numel // _NWORKERS
    nwin = shard // _WIN
    hwords = 16 * nbuckets
    idx_mask = (nbuckets - 1) << shift
    mesh = plsc.VectorSubcoreMesh(
        core_axis_name="c", subcore_axis_name="s", num_cores=2, num_subcores=16
    )

    def body(x_hbm, *rest):
        if filt_shift is not None:
            sel_hbm, hist_hbm, wbuf, hist_v, scal_v, sem0, sem1 = rest
        else:
            hist_hbm, wbuf, hist_v, scal_v, sem0, sem1 = rest
        sems = (sem0, sem1)
        cid = lax.axis_index("c")
        sid = lax.axis_index("s")
        wid = sid * 2 + cid
        base = wid * shard

        zero16 = jnp.zeros((16,), jnp.int32)

        def zbody(i, _):
            hist_v[pl.ds(i * 16, 16)] = zero16
            return 0

        lax.fori_loop(0, hwords // 16, zbody, 0, unroll=8)

        if filt_shift is not None:
            pltpu.sync_copy(sel_hbm, scal_v)
            selv = scal_v[...]

        lane = lax.iota(jnp.int32, 16)
        ones = jnp.ones((16,), jnp.int32)

        def start_copy(g, b):
            pltpu.make_async_copy(
                x_hbm.at[pl.ds(base + g * _WIN, _WIN)], wbuf.at[b], sems[b]
            ).start()

        def wait_copy(g, b):
            pltpu.make_async_copy(
                x_hbm.at[pl.ds(base + g * _WIN, _WIN)], wbuf.at[b], sems[b]
            ).wait()

        def window(g, b):
            @pl.when(g + 1 < nwin)
            def _():
                start_copy(g + 1, 1 - b)

            wait_copy(g, b)

            @plsc.parallel_loop(0, _WIN, step=16, unroll=8)
            def _(off):
                raw = wbuf[b, pl.ds(off, 16)]
                masked = raw & jnp.int32(idx_mask)
                if shift >= 4:
                    bkt16 = lax.shift_right_logical(masked, shift - 4)
                else:
                    bkt16 = lax.shift_left(masked, 4 - shift)
                idx = bkt16 | lane
                if filt_shift is not None:
                    bits = raw & jnp.int32(_ABS_MASK)
                    m = lax.shift_right_logical(bits, filt_shift) == selv
                    plsc.addupdate_scatter(hist_v, [idx], ones, mask=m)
                else:
                    plsc.addupdate_scatter(hist_v, [idx], ones)

        def pair(gp, _):
            for b in range(2):
                window(gp * 2 + b, b)
            return 0

        start_copy(0, 0)
        lax.fori_loop(0, nwin // 2, pair, 0)
        pltpu.sync_copy(hist_v, hist_hbm.at[wid])

    return functools.partial(
        pl.kernel,
        body,
        out_type=jax.ShapeDtypeStruct((_NWORKERS, hwords), jnp.int32),
        mesh=mesh,
        compiler_params=pltpu.CompilerParams(needs_layout_passes=False),
        scratch_types=[
            pltpu.VMEM((2, _WIN), jnp.int32),
            pltpu.VMEM((hwords,), jnp.int32),
            pltpu.VMEM((16,), jnp.int32),
            pltpu.SemaphoreType.DMA,
            pltpu.SemaphoreType.DMA,
        ],
    )()


def _make_merge(nbuckets, shift_this, k_static):
    nbits = nbuckets.bit_length() - 1
    has_prev = k_static is None

    def body(*refs):
        if has_prev:
            psel_ref, pkrem_ref, hist_ref, sel_out, krem_out = refs
            prev_sel = psel_ref[0]
            k_t = pkrem_ref[0]
        else:
            hist_ref, sel_out, krem_out = refs
            prev_sel = jnp.int32(0)
            k_t = jnp.int32(k_static)
        h = jnp.sum(hist_ref[...], axis=0, keepdims=True)
        iot = lax.broadcasted_iota(jnp.int32, (1, nbuckets * 16), 1)
        lo = jnp.int32(0)
        hi = jnp.int32(nbuckets)
        for _ in range(nbits):
            mid = (lo + hi) // 2
            c = jnp.sum(jnp.where(iot >= mid * 16, h, 0))
            ge = c >= k_t
            lo = jnp.where(ge, mid, lo)
            hi = jnp.where(ge, hi, mid)
        c_above = jnp.sum(jnp.where(iot >= (lo + 1) * 16, h, 0))
        new_sel = (prev_sel * jnp.int32(nbuckets)) + lo
        for i in range(16):
            sel_out[i] = new_sel
        krem_out[0] = k_t - c_above

    in_specs = [pl.BlockSpec(memory_space=pltpu.VMEM)]
    if has_prev:
        in_specs = [
            pl.BlockSpec(memory_space=pltpu.SMEM),
            pl.BlockSpec(memory_space=pltpu.SMEM),
        ] + in_specs
    return pl.pallas_call(
        body,
        in_specs=in_specs,
        out_specs=[
            pl.BlockSpec(memory_space=pltpu.SMEM),
            pl.BlockSpec(memory_space=pltpu.SMEM),
        ],
        out_shape=[
            jax.ShapeDtypeStruct((16,), jnp.int32),
            jax.ShapeDtypeStruct((1,), jnp.int32),
        ],
    )


def _mask_body(t_ref, x_ref, o_ref):
    t = t_ref[0]
    xv = x_ref[...]
    bits = lax.bitcast_convert_type(xv, jnp.int32) & jnp.int32(_ABS_MASK)
    o_ref[...] = jnp.where(bits >= t, xv, 0.0)


def kernel(x, bias):
    rows, cols = x.shape
    numel = rows * cols
    k = max(1, int(round(numel * (1.0 - _sparsity()))))
    xf = lax.bitcast_convert_type(x, jnp.int32).reshape(-1)

    hist1 = _make_sc_hist(numel, 4096, 19, None)(xf)
    sel1, krem1 = _make_merge(4096, 12, k)(hist1)

    hist2 = _make_sc_hist(numel, 4096, 7, 19)(xf, sel1)
    sel2, krem2 = _make_merge(4096, 12, None)(sel1, krem1, hist2)

    hist3 = _make_sc_hist(numel, 128, 0, 7)(xf, sel2)
    sel3, _ = _make_merge(128, 7, None)(sel2, krem2, hist3)

    nchunks = 8
    blk = rows // nchunks
    masked = pl.pallas_call(
        _mask_body,
        grid=(nchunks,),
        in_specs=[
            pl.BlockSpec(memory_space=pltpu.SMEM),
            pl.BlockSpec((blk, cols), lambda c: (c, 0)),
        ],
        out_specs=pl.BlockSpec((blk, cols), lambda c: (c, 0)),
        out_shape=jax.ShapeDtypeStruct((rows, cols), jnp.float32),
    )(sel3, x)

    return (masked, bias)


# speedup vs baseline: 4.7600x; 1.6886x over previous
import functools

import jax
import jax.numpy as jnp
from jax import lax
from jax.experimental import pallas as pl
from jax.experimental.pallas import tpu as pltpu
from jax.experimental.pallas import tpu_sc as plsc

_ABS_MASK = 0x7FFFFFFF
_NWORKERS = 32
_WIN = 8192


def _sparsity() -> float:
    s_i, s_f = 0.5, 0.9
    return s_f + (s_i - s_f) * (1.0 - 0.0) ** 3


_WROWS = 8


def _make_sc_hist(rows, cols, nbuckets, shift, filt_shift):
    wrows = rows // _NWORKERS
    nwin = wrows // _WROWS
    hwords = 16 * nbuckets
    idx_mask = (nbuckets - 1) << shift
    mesh = plsc.VectorSubcoreMesh(
        core_axis_name="c", subcore_axis_name="s", num_cores=2, num_subcores=16
    )

    def body(x_hbm, *rest):
        if filt_shift is not None:
            sel_hbm, hist_hbm, wbuf, hist_v, scal_v, sem0, sem1 = rest
        else:
            hist_hbm, wbuf, hist_v, scal_v, sem0, sem1 = rest
        sems = (sem0, sem1)
        cid = lax.axis_index("c")
        sid = lax.axis_index("s")
        wid = sid * 2 + cid
        base = wid * wrows

        zero16 = jnp.zeros((16,), jnp.int32)

        def zbody(i, _):
            hist_v[pl.ds(i * 16, 16)] = zero16
            return 0

        lax.fori_loop(0, hwords // 16, zbody, 0, unroll=8)

        if filt_shift is not None:
            pltpu.sync_copy(sel_hbm, scal_v)
            selv = scal_v[...]

        lane = lax.iota(jnp.int32, 16)
        ones = jnp.ones((16,), jnp.int32)

        def start_copy(g, b):
            pltpu.make_async_copy(
                x_hbm.at[pl.ds(base + g * _WROWS, _WROWS)], wbuf.at[b], sems[b]
            ).start()

        def wait_copy(g, b):
            pltpu.make_async_copy(
                x_hbm.at[pl.ds(base + g * _WROWS, _WROWS)], wbuf.at[b], sems[b]
            ).wait()

        def window(g, b):
            @pl.when(g + 1 < nwin)
            def _():
                start_copy(g + 1, 1 - b)

            wait_copy(g, b)

            @plsc.parallel_loop(0, cols, step=16, unroll=2)
            def _(off):
                for r in range(_WROWS):
                    raw = plsc.bitcast(wbuf[b, r, pl.ds(off, 16)], jnp.int32)
                    masked = raw & jnp.int32(idx_mask)
                    if shift >= 4:
                        bkt16 = lax.shift_right_logical(masked, shift - 4)
                    else:
                        bkt16 = lax.shift_left(masked, 4 - shift)
                    idx = bkt16 | lane
                    if filt_shift is not None:
                        bits = raw & jnp.int32(_ABS_MASK)
                        m = lax.shift_right_logical(bits, filt_shift) == selv
                        plsc.addupdate_scatter(hist_v, [idx], ones, mask=m)
                    else:
                        plsc.addupdate_scatter(hist_v, [idx], ones)

        def pair(gp, _):
            for b in range(2):
                window(gp * 2 + b, b)
            return 0

        start_copy(0, 0)
        lax.fori_loop(0, nwin // 2, pair, 0)
        pltpu.sync_copy(hist_v, hist_hbm.at[wid])

    return functools.partial(
        pl.kernel,
        body,
        out_type=jax.ShapeDtypeStruct((_NWORKERS, hwords), jnp.int32),
        mesh=mesh,
        compiler_params=pltpu.CompilerParams(needs_layout_passes=False),
        scratch_types=[
            pltpu.VMEM((2, _WROWS, cols), jnp.float32),
            pltpu.VMEM((hwords,), jnp.int32),
            pltpu.VMEM((16,), jnp.int32),
            pltpu.SemaphoreType.DMA,
            pltpu.SemaphoreType.DMA,
        ],
    )()


def _make_merge(nbuckets, shift_this, k_static):
    nbits = nbuckets.bit_length() - 1
    has_prev = k_static is None

    def body(*refs):
        if has_prev:
            psel_ref, pkrem_ref, hist_ref, sel_out, krem_out = refs
            prev_sel = psel_ref[0]
            k_t = pkrem_ref[0]
        else:
            hist_ref, sel_out, krem_out = refs
            prev_sel = jnp.int32(0)
            k_t = jnp.int32(k_static)
        h = jnp.sum(hist_ref[...], axis=0, keepdims=True)
        iot = lax.broadcasted_iota(jnp.int32, (1, nbuckets * 16), 1)
        lo = jnp.int32(0)
        hi = jnp.int32(nbuckets)
        for _ in range(nbits):
            mid = (lo + hi) // 2
            c = jnp.sum(jnp.where(iot >= mid * 16, h, 0))
            ge = c >= k_t
            lo = jnp.where(ge, mid, lo)
            hi = jnp.where(ge, hi, mid)
        c_above = jnp.sum(jnp.where(iot >= (lo + 1) * 16, h, 0))
        new_sel = (prev_sel * jnp.int32(nbuckets)) + lo
        for i in range(16):
            sel_out[i] = new_sel
        krem_out[0] = k_t - c_above

    in_specs = [pl.BlockSpec(memory_space=pltpu.VMEM)]
    if has_prev:
        in_specs = [
            pl.BlockSpec(memory_space=pltpu.SMEM),
            pl.BlockSpec(memory_space=pltpu.SMEM),
        ] + in_specs
    return pl.pallas_call(
        body,
        in_specs=in_specs,
        out_specs=[
            pl.BlockSpec(memory_space=pltpu.SMEM),
            pl.BlockSpec(memory_space=pltpu.SMEM),
        ],
        out_shape=[
            jax.ShapeDtypeStruct((16,), jnp.int32),
            jax.ShapeDtypeStruct((1,), jnp.int32),
        ],
    )


def _mask_body(t_ref, x_ref, o_ref):
    t = t_ref[0]
    xv = x_ref[...]
    bits = lax.bitcast_convert_type(xv, jnp.int32) & jnp.int32(_ABS_MASK)
    o_ref[...] = jnp.where(bits >= t, xv, 0.0)


def kernel(x, bias):
    rows, cols = x.shape
    numel = rows * cols
    k = max(1, int(round(numel * (1.0 - _sparsity()))))

    hist1 = _make_sc_hist(rows, cols, 2048, 20, None)(x)
    sel1, krem1 = _make_merge(2048, 20, k)(hist1)

    hist2 = _make_sc_hist(rows, cols, 2048, 9, 20)(x, sel1)
    sel2, krem2 = _make_merge(2048, 9, None)(sel1, krem1, hist2)

    hist3 = _make_sc_hist(rows, cols, 512, 0, 9)(x, sel2)
    sel3, _ = _make_merge(512, 0, None)(sel2, krem2, hist3)

    nchunks = 8
    blk = rows // nchunks
    masked = pl.pallas_call(
        _mask_body,
        grid=(nchunks,),
        in_specs=[
            pl.BlockSpec(memory_space=pltpu.SMEM),
            pl.BlockSpec((blk, cols), lambda c: (c, 0)),
        ],
        out_specs=pl.BlockSpec((blk, cols), lambda c: (c, 0)),
        out_shape=jax.ShapeDtypeStruct((rows, cols), jnp.float32),
    )(sel3, x)

    return (masked, bias)
